# baseline (speedup 1.0000x reference)
import functools
import numpy as np
import jax
import jax.numpy as jnp
from jax import lax
from jax.experimental import pallas as pl
from jax.experimental.pallas import tpu as pltpu, tpu_sc as plsc

_N_ANCH = 48960
_N_TILES = 383
_TPW = 12

_SQ = [float(np.float32(np.sqrt(r))) for r in (0.5, 1.0, 2.0)]
_ISQ = [float(np.float32(1.0 / np.sqrt(r))) for r in (0.5, 1.0, 2.0)]
_SC = [float(np.float32(2.0 ** (k / 3.0))) for k in range(3)]

_TB = (288, 360, 378)
_SBASE = (0.0, 4096.0, 5120.0, 5376.0)
_STRIDE = (8.0, 16.0, 32.0, 64.0)
_WMASK = (63, 31, 15, 7)
_INVW = (1.0 / 64, 1.0 / 32, 1.0 / 16, 1.0 / 8)
_SIZE = (32.0, 64.0, 128.0, 256.0)


def _by_tile(t, vals, dtype):
    v = jnp.asarray(vals[3], dtype)
    v = jnp.where(t < _TB[2], jnp.asarray(vals[2], dtype), v)
    v = jnp.where(t < _TB[1], jnp.asarray(vals[1], dtype), v)
    v = jnp.where(t < _TB[0], jnp.asarray(vals[0], dtype), v)
    return v


def _make():
    mesh = plsc.VectorSubcoreMesh(core_axis_name="c", subcore_axis_name="s")

    @functools.partial(
        pl.kernel,
        mesh=mesh,
        out_type=(
            jax.ShapeDtypeStruct((_N_TILES, 4, 128), jnp.float32),
            jax.ShapeDtypeStruct((_N_TILES, 4, 128), jnp.float32),
        ),
        scratch_types=[
            pltpu.VMEM((_TPW, 4, 128), jnp.float32),
            pltpu.VMEM((_TPW, 4, 128), jnp.float32),
        ],
    )
    def k(xywh_hbm, xyxy_hbm, bwh, bxy):
        wid = lax.axis_index("s") * 2 + lax.axis_index("c")
        lane = lax.iota(jnp.int32, 16)

        def tile_body(kk, cc):
            t = wid * _TPW + kk
            m = t * 128
            site_t = m // 9
            a_t = m - site_t * 9
            sbase = _by_tile(t, _SBASE, jnp.float32)
            stride = _by_tile(t, _STRIDE, jnp.float32)
            wmask = _by_tile(t, _WMASK, jnp.int32)
            invw = _by_tile(t, _INVW, jnp.float32)
            size = _by_tile(t, _SIZE, jnp.float32)
            sloc_t = site_t.astype(jnp.float32) - sbase

            for v in range(8):
                off = a_t + v * 16
                qv = off // 9
                rv = off - qv * 9
                r = rv + lane
                q2 = jnp.where(r >= 18, 2, jnp.where(r >= 9, 1, 0))
                a = r - q2 * 9
                sloc = (sloc_t + (qv + q2).astype(jnp.float32))
                sloci = site_t + qv + q2 - sbase.astype(jnp.int32)
                x = sloci & wmask
                xf = x.astype(jnp.float32)
                yf = (sloc - xf) * invw
                cx = (xf + 0.5) * stride
                cy = (yf + 0.5) * stride

                ri = jnp.where(a >= 6, 2, jnp.where(a >= 3, 1, 0))
                si = a - ri * 3
                scale = jnp.where(si == 1, _SC[1], jnp.where(si == 2, _SC[2], _SC[0]))
                sq = jnp.where(ri == 1, _SQ[1], jnp.where(ri == 2, _SQ[2], _SQ[0]))
                isq = jnp.where(ri == 1, _ISQ[1], jnp.where(ri == 2, _ISQ[2], _ISQ[0]))
                wa = size * (scale * sq)
                ha = size * (scale * isq)

                ds = pl.ds(v * 16, 16)
                bwh[kk, 0, ds] = cx
                bwh[kk, 1, ds] = cy
                bwh[kk, 2, ds] = wa
                bwh[kk, 3, ds] = ha
                hw = wa * 0.5
                hh = ha * 0.5
                bxy[kk, 0, ds] = cx - hw
                bxy[kk, 1, ds] = cy - hh
                bxy[kk, 2, ds] = cx + hw
                bxy[kk, 3, ds] = cy + hh
            return cc

        lax.fori_loop(0, _TPW, tile_body, jnp.int32(0), unroll=False)

        @pl.when(wid < 31)
        def _():
            pltpu.sync_copy(bwh, xywh_hbm.at[pl.ds(wid * _TPW, _TPW)])
            pltpu.sync_copy(bxy, xyxy_hbm.at[pl.ds(wid * _TPW, _TPW)])

        @pl.when(wid == 31)
        def _():
            pltpu.sync_copy(bwh.at[pl.ds(0, 11)], xywh_hbm.at[pl.ds(372, 11)])
            pltpu.sync_copy(bxy.at[pl.ds(0, 11)], xyxy_hbm.at[pl.ds(372, 11)])

    return k


def kernel(feat0, feat1, feat2, feat3):
    a, b = _make()()
    a4 = a.transpose(0, 2, 1).reshape(_N_TILES * 128, 4)[:_N_ANCH]
    b4 = b.transpose(0, 2, 1).reshape(_N_TILES * 128, 4)[:_N_ANCH]
    return (a4, b4)


# speedup vs baseline: 7.6207x; 7.6207x over previous
import numpy as np
import jax
import jax.numpy as jnp
from jax.experimental import pallas as pl

_STRIDES = (8, 16, 32, 64)
_SIZES = (32, 64, 128, 256)
_HW = (64, 32, 16, 8)
_RATIOS = np.array([0.5, 1.0, 2.0])
_SCALES = np.array([1.0, 2.0 ** (1.0 / 3.0), 2.0 ** (2.0 / 3.0)])
_A = 9
_W = 1152
_N_ANCH = sum(h * h * _A for h in _HW)


def _wh_table(box_size):
    anchors = box_size * np.tile(_SCALES, (2, len(_RATIOS))).T
    areas = anchors[:, 0] * anchors[:, 1]
    anchors[:, 0] = np.sqrt(areas * np.repeat(_RATIOS, len(_SCALES)))
    anchors[:, 1] = anchors[:, 0] / np.repeat(_RATIOS, len(_SCALES))
    return anchors.astype(np.float32)


def _gen_body(xywh_ref, xyxy_ref):
    c = jax.lax.broadcasted_iota(jnp.int32, (4, 1), 0)
    is_cx = c == 0
    is_cy = c == 1
    is_w = c == 2
    is_x = c % 2 == 0
    is_cy_f = is_cy.astype(jnp.float32)
    is_y_f = (c % 2 == 1).astype(jnp.float32)

    def build(n, hw, stride, size):
        s = float(stride)
        tab = _wh_table(size)
        i = jax.lax.broadcasted_iota(jnp.int32, (1, n), 1)
        site = i // _A
        a = i - site * _A
        x = site & (hw - 1)
        y = site >> hw.bit_length() - 1
        cx = (x.astype(jnp.float32) + 0.5) * s
        cy = (y.astype(jnp.float32) + 0.5) * s
        wa = jnp.full(i.shape, float(tab[0, 0]), jnp.float32)
        ha = jnp.full(i.shape, float(tab[0, 1]), jnp.float32)
        for k in range(1, _A):
            sel = a == k
            wa = jnp.where(sel, float(tab[k, 0]), wa)
            ha = jnp.where(sel, float(tab[k, 1]), ha)
        xywh = jnp.where(is_cx, cx, jnp.where(is_cy, cy, jnp.where(is_w, wa, ha)))
        ctr = jnp.where(is_x, cx, cy)
        half = jnp.where(is_x, wa, ha) * 0.5
        xyxy = jnp.where(c < 2, ctr - half, ctr + half)
        return xywh, xyxy

    off = 0
    for hw, stride, size in zip(_HW, _STRIDES, _SIZES):
        n = hw * hw * _A
        if n > _W:
            pwh, pxy = build(_W, hw, stride, size)
            reps = n // _W
            twh = jnp.tile(pwh, (1, reps))
            txy = jnp.tile(pxy, (1, reps))
            i = jax.lax.broadcasted_iota(jnp.int32, (1, n), 1)
            rep = (i >> 7) // _A
            step = float(stride) * (_W // (_A * hw))
            rep_f = rep.astype(jnp.float32) * step
            xywh = twh + is_cy_f * rep_f
            xyxy = txy + is_y_f * rep_f
        else:
            xywh, xyxy = build(n, hw, stride, size)
        xywh_ref[:, pl.ds(off, n)] = xywh
        xyxy_ref[:, pl.ds(off, n)] = xyxy
        off += n


def _generate():
    out_shape = (
        jax.ShapeDtypeStruct((4, _N_ANCH), jnp.float32),
        jax.ShapeDtypeStruct((4, _N_ANCH), jnp.float32),
    )
    return pl.pallas_call(_gen_body, out_shape=out_shape)()


def kernel(feat0, feat1, feat2, feat3):
    xywh_t, xyxy_t = _generate()
    return (xywh_t.T, xyxy_t.T)


# speedup vs baseline: 11.3872x; 1.4942x over previous
import numpy as np
import jax
import jax.numpy as jnp
from jax.experimental import pallas as pl

_STRIDES = (8, 16, 32, 64)
_SIZES = (32, 64, 128, 256)
_HW = (64, 32, 16, 8)
_RATIOS = np.array([0.5, 1.0, 2.0])
_SCALES = np.array([1.0, 2.0 ** (1.0 / 3.0), 2.0 ** (2.0 / 3.0)])
_A = 9
_W = 1152
_N_ANCH = sum(h * h * _A for h in _HW)


def _wh_table(box_size):
    anchors = box_size * np.tile(_SCALES, (2, len(_RATIOS))).T
    areas = anchors[:, 0] * anchors[:, 1]
    anchors[:, 0] = np.sqrt(areas * np.repeat(_RATIOS, len(_SCALES)))
    anchors[:, 1] = anchors[:, 0] / np.repeat(_RATIOS, len(_SCALES))
    return anchors.astype(np.float32)


def _gen_body(xywh_ref, xyxy_ref):
    c = jax.lax.broadcasted_iota(jnp.int32, (4, 1), 0)
    is_cx = c == 0
    is_cy = c == 1
    is_w = c == 2
    is_x = c % 2 == 0
    is_cy_f = is_cy.astype(jnp.float32)
    is_y_f = (c % 2 == 1).astype(jnp.float32)

    def build(n, hw, stride, size):
        s = float(stride)
        tab = _wh_table(size)
        i = jax.lax.broadcasted_iota(jnp.int32, (1, n), 1)
        site = i // _A
        a = i - site * _A
        x = site & (hw - 1)
        y = site >> hw.bit_length() - 1
        cx = (x.astype(jnp.float32) + 0.5) * s
        cy = (y.astype(jnp.float32) + 0.5) * s
        wa = jnp.full(i.shape, float(tab[0, 0]), jnp.float32)
        ha = jnp.full(i.shape, float(tab[0, 1]), jnp.float32)
        for k in range(1, _A):
            sel = a == k
            wa = jnp.where(sel, float(tab[k, 0]), wa)
            ha = jnp.where(sel, float(tab[k, 1]), ha)
        xywh = jnp.where(is_cx, cx, jnp.where(is_cy, cy, jnp.where(is_w, wa, ha)))
        ctr = jnp.where(is_x, cx, cy)
        half = jnp.where(is_x, wa, ha) * 0.5
        xyxy = jnp.where(c < 2, ctr - half, ctr + half)
        return xywh, xyxy

    off = 0
    for hw, stride, size in zip(_HW, _STRIDES, _SIZES):
        n = hw * hw * _A
        if n > _W:
            pwh, pxy = build(_W, hw, stride, size)
            step = float(stride) * (_W // (_A * hw))
            awh = is_cy_f * step
            axy = is_y_f * step
            for r in range(n // _W):
                ds = pl.ds(off + r * _W, _W)
                if r == 0:
                    xywh_ref[:, ds] = pwh
                    xyxy_ref[:, ds] = pxy
                else:
                    xywh_ref[:, ds] = pwh + awh * float(r)
                    xyxy_ref[:, ds] = pxy + axy * float(r)
        else:
            xywh, xyxy = build(n, hw, stride, size)
            xywh_ref[:, pl.ds(off, n)] = xywh
            xyxy_ref[:, pl.ds(off, n)] = xyxy
        off += n


def _generate():
    out_shape = (
        jax.ShapeDtypeStruct((4, _N_ANCH), jnp.float32),
        jax.ShapeDtypeStruct((4, _N_ANCH), jnp.float32),
    )
    return pl.pallas_call(_gen_body, out_shape=out_shape)()


def kernel(feat0, feat1, feat2, feat3):
    xywh_t, xyxy_t = _generate()
    return (xywh_t.T, xyxy_t.T)
